# baseline (speedup 1.0000x reference)
import functools

import jax
import jax.numpy as jnp
from jax import lax
from jax.experimental import pallas as pl
from jax.experimental.pallas import tpu as pltpu
from jax.experimental.pallas import tpu_sc as plsc

_SIZE = 3
_COLS = 1024
_NW = 8
_CPW = _COLS // _NW
_LANES = 16

_mesh = plsc.VectorSubcoreMesh(
    core_axis_name="c", subcore_axis_name="s", num_cores=1
)


@functools.partial(
    pl.kernel,
    mesh=_mesh,
    out_type=jax.ShapeDtypeStruct((_SIZE, _COLS), jnp.float32),
    scratch_types=[
        pltpu.VMEM((_SIZE, _SIZE, _LANES), jnp.float32),
        pltpu.VMEM((_SIZE, _CPW), jnp.float32),
        pltpu.VMEM((_SIZE, _CPW), jnp.float32),
        pltpu.SemaphoreType.DMA,
    ],
)
def _spmv(v_hbm, m_hbm, out_hbm, m_v, v_v, o_v, sem):
    wid = lax.axis_index("s")

    @pl.when(wid < _NW)
    def _():
        base = wid * _CPW
        c_m = pltpu.async_copy(m_hbm, m_v, sem)
        c_v = pltpu.async_copy(v_hbm.at[:, pl.ds(base, _CPW)], v_v, sem)
        c_m.wait()
        c_v.wait()
        m = [[m_v[r, k, :] for k in range(_SIZE)] for r in range(_SIZE)]

        def _col_block(j, carry):
            sl = pl.ds(j * _LANES, _LANES)
            rows = [v_v[k, sl] for k in range(_SIZE)]
            for r in range(_SIZE):
                acc = m[r][0] * rows[0]
                for k in range(1, _SIZE):
                    acc = acc + m[r][k] * rows[k]
                o_v[r, sl] = acc
            return carry

        lax.fori_loop(0, _CPW // _LANES, _col_block, 0)
        pltpu.async_copy(o_v, out_hbm.at[:, pl.ds(base, _CPW)], sem).wait()


def kernel(v, M_hat):
    m_b = jnp.broadcast_to(M_hat[:, :, None], (_SIZE, _SIZE, _LANES))
    return _spmv(v, m_b)


# speedup vs baseline: 1.0225x; 1.0225x over previous
import functools

import jax
import jax.numpy as jnp
from jax import lax
from jax.experimental import pallas as pl
from jax.experimental.pallas import tpu as pltpu
from jax.experimental.pallas import tpu_sc as plsc

_SIZE = 3
_COLS = 1024
_NW = 8
_CPW = _COLS // _NW
_LANES = 16

_mesh = plsc.VectorSubcoreMesh(
    core_axis_name="c", subcore_axis_name="s", num_cores=1
)


@functools.partial(
    pl.kernel,
    mesh=_mesh,
    out_type=jax.ShapeDtypeStruct((_SIZE, _COLS), jnp.float32),
    scratch_types=[
        pltpu.VMEM((_SIZE, _SIZE, _LANES), jnp.float32),
        pltpu.VMEM((_SIZE, _CPW), jnp.float32),
        pltpu.VMEM((_SIZE, _CPW), jnp.float32),
        pltpu.SemaphoreType.DMA,
    ],
)
def _spmv(v_hbm, m_hbm, out_hbm, m_v, v_v, o_v, sem):
    wid = lax.axis_index("s")

    @pl.when(wid < _NW)
    def _():
        base = wid * _CPW
        c_m = pltpu.async_copy(m_hbm, m_v, sem)
        c_v = pltpu.async_copy(v_hbm.at[:, pl.ds(base, _CPW)], v_v, sem)
        c_m.wait()
        c_v.wait()
        m = [[m_v[r, k, :] for k in range(_SIZE)] for r in range(_SIZE)]
        for j in range(_CPW // _LANES):
            sl = pl.ds(j * _LANES, _LANES)
            rows = [v_v[k, sl] for k in range(_SIZE)]
            for r in range(_SIZE):
                acc = m[r][0] * rows[0]
                for k in range(1, _SIZE):
                    acc = acc + m[r][k] * rows[k]
                o_v[r, sl] = acc
        pltpu.async_copy(o_v, out_hbm.at[:, pl.ds(base, _CPW)], sem).wait()


def kernel(v, M_hat):
    m_b = jnp.broadcast_to(M_hat[:, :, None], (_SIZE, _SIZE, _LANES))
    return _spmv(v, m_b)
